# baseline (speedup 1.0000x reference)
import functools

import jax
import jax.numpy as jnp
from jax import lax
from jax.experimental import pallas as pl
from jax.experimental.pallas import tpu as pltpu
from jax.experimental.pallas import tpu_sc as plsc

_TOP_K = 2
_ALPHA = 0.01
_TB = 512
_NC, _NS, _L = 2, 16, 16
_NW = _NC * _NS


def _gate_body(x_ref, w_ref, probs_ref, stats_ref):
    step = pl.program_id(0)
    x = x_ref[...]
    w = w_ref[...]
    logits = lax.dot_general(
        x, w, (((1,), (1,)), ((), ())), preferred_element_type=jnp.float32
    )
    m1 = jnp.max(logits, axis=-1, keepdims=True)
    ex = jnp.exp(logits - m1)
    z = jnp.sum(ex, axis=-1, keepdims=True)
    probs = ex / z
    probs_ref[...] = probs

    @pl.when(step == 0)
    def _init():
        stats_ref[...] = jnp.zeros_like(stats_ref)

    stats_ref[0:1, :] += jnp.sum(probs, axis=0, keepdims=True)


def _route_body(probs_hbm, wout_hbm, iout_hbm, cnt_hbm, pv, wv, iv, cv,
                *, n_experts, tokens_per_worker):
    wid = lax.axis_index("s") * _NC + lax.axis_index("c")
    base = wid * tokens_per_worker
    pltpu.sync_copy(
        probs_hbm.at[pl.ds(base * n_experts, tokens_per_worker * n_experts)], pv)

    for j in range(n_experts // _L):
        cv[pl.ds(j * _L, _L)] = jnp.zeros((_L,), jnp.float32)

    iota = lax.iota(jnp.int32, _L)
    ones = jnp.ones((_L,), jnp.float32)

    def group(g, carry):
        row = g * _L + iota
        pbase = row * n_experts
        m1 = plsc.load_gather(pv, [pbase])
        i1 = jnp.zeros((_L,), jnp.int32)
        m2 = jnp.full((_L,), -1.0, jnp.float32)
        i2 = jnp.zeros((_L,), jnp.int32)
        for e in range(1, n_experts):
            v = plsc.load_gather(pv, [pbase + e])
            gt1 = v > m1
            gt2 = v > m2
            m2 = jnp.where(gt1, m1, jnp.where(gt2, v, m2))
            i2 = jnp.where(gt1, i1, jnp.where(gt2, e, i2))
            m1 = jnp.where(gt1, v, m1)
            i1 = jnp.where(gt1, e, i1)
        denom = m1 + m2 + 1e-9
        obase = row * _TOP_K
        plsc.store_scatter(wv, [obase], m1 / denom)
        plsc.store_scatter(wv, [obase + 1], m2 / denom)
        plsc.store_scatter(iv, [obase], i1)
        plsc.store_scatter(iv, [obase + 1], i2)
        plsc.addupdate_scatter(cv, [i1], ones)
        plsc.addupdate_scatter(cv, [i2], ones)
        return carry

    lax.fori_loop(0, tokens_per_worker // _L, group, 0)

    pltpu.sync_copy(wv, wout_hbm.at[pl.ds(base * _TOP_K, tokens_per_worker * _TOP_K)])
    pltpu.sync_copy(iv, iout_hbm.at[pl.ds(base * _TOP_K, tokens_per_worker * _TOP_K)])
    pltpu.sync_copy(cv, cnt_hbm.at[wid])


def _aux_body(stats_ref, cnt_ref, aux_ref, *, n_tokens, n_experts):
    p_mean = stats_ref[0:1, :] / n_tokens
    cnt = jnp.sum(cnt_ref[...], axis=0, keepdims=True)
    f_mean = cnt / (n_tokens * _TOP_K)
    aux = _ALPHA * n_experts * jnp.sum(p_mean * f_mean)
    aux_ref[...] = jnp.broadcast_to(aux, (1, n_experts))


def kernel(hidden_states, gate_weight):
    b, s, h = hidden_states.shape
    e = gate_weight.shape[0]
    t = b * s
    x = hidden_states.reshape(t, h)

    probs, stats = pl.pallas_call(
        _gate_body,
        grid=(t // _TB,),
        in_specs=[
            pl.BlockSpec((_TB, h), lambda i: (i, 0)),
            pl.BlockSpec((e, h), lambda i: (0, 0)),
        ],
        out_specs=[
            pl.BlockSpec((_TB, e), lambda i: (i, 0)),
            pl.BlockSpec((8, e), lambda i: (0, 0)),
        ],
        out_shape=[
            jax.ShapeDtypeStruct((t, e), jnp.float32),
            jax.ShapeDtypeStruct((8, e), jnp.float32),
        ],
    )(x, gate_weight)

    tpw = t // _NW
    mesh = plsc.VectorSubcoreMesh(
        core_axis_name="c", subcore_axis_name="s",
        num_cores=_NC, num_subcores=_NS,
    )
    route = pl.kernel(
        functools.partial(_route_body, n_experts=e, tokens_per_worker=tpw),
        out_type=[
            jax.ShapeDtypeStruct((t * _TOP_K,), jnp.float32),
            jax.ShapeDtypeStruct((t * _TOP_K,), jnp.int32),
            jax.ShapeDtypeStruct((_NW, e), jnp.float32),
        ],
        mesh=mesh,
        compiler_params=pltpu.CompilerParams(needs_layout_passes=False),
        scratch_types=[
            pltpu.VMEM((tpw * e,), jnp.float32),
            pltpu.VMEM((tpw * _TOP_K,), jnp.float32),
            pltpu.VMEM((tpw * _TOP_K,), jnp.int32),
            pltpu.VMEM((e,), jnp.float32),
        ],
    )
    wout, iout, cnt = route(probs.reshape(t * e))

    aux = pl.pallas_call(
        functools.partial(_aux_body, n_tokens=t, n_experts=e),
        in_specs=[
            pl.BlockSpec((8, e), lambda: (0, 0)),
            pl.BlockSpec((_NW, e), lambda: (0, 0)),
        ],
        out_specs=pl.BlockSpec((1, e), lambda: (0, 0)),
        out_shape=jax.ShapeDtypeStruct((1, e), jnp.float32),
    )(stats, cnt)

    return (
        wout.reshape(b, s, _TOP_K),
        iout.reshape(b, s, _TOP_K).astype(jnp.int64),
        aux[0, 0],
    )


# speedup vs baseline: 1.7953x; 1.7953x over previous
import functools

import jax
import jax.numpy as jnp
from jax import lax
from jax.experimental import pallas as pl
from jax.experimental.pallas import tpu as pltpu

_TOP_K = 2
_ALPHA = 0.01
_TB = 512


def _router_body(x_ref, w_ref, wout_ref, iout_ref, stats_ref, p_acc, c_acc,
                 *, n_tokens, n_experts):
    step = pl.program_id(0)
    nsteps = pl.num_programs(0)
    x = x_ref[...]
    w = w_ref[...]
    logits = lax.dot_general(
        w, x, (((1,), (1,)), ((), ())), preferred_element_type=jnp.float32
    )

    eidx = lax.broadcasted_iota(jnp.int32, logits.shape, 0)
    m1 = jnp.max(logits, axis=0, keepdims=True)
    i1 = jnp.min(jnp.where(logits == m1, eidx, n_experts), axis=0, keepdims=True)
    masked = jnp.where(eidx == i1, -jnp.inf, logits)
    m2 = jnp.max(masked, axis=0, keepdims=True)
    i2 = jnp.min(jnp.where(masked == m2, eidx, n_experts), axis=0, keepdims=True)

    ex = jnp.exp(logits - m1)
    z = jnp.sum(ex, axis=0, keepdims=True)
    p1 = 1.0 / z
    p2 = jnp.exp(m2 - m1) / z
    denom = p1 + p2 + 1e-9
    wout_ref[...] = jnp.concatenate([p1 / denom, p2 / denom], axis=0).T
    iout_ref[...] = jnp.concatenate([i1, i2], axis=0).T

    one_hot = (eidx == i1).astype(jnp.float32) + (eidx == i2).astype(jnp.float32)

    @pl.when(step == 0)
    def _init():
        p_acc[...] = jnp.zeros_like(p_acc)
        c_acc[...] = jnp.zeros_like(c_acc)

    p_acc[...] += ex / z
    c_acc[...] += one_hot

    @pl.when(step == nsteps - 1)
    def _finish():
        p_mean = jnp.sum(p_acc[...], axis=1, keepdims=True) / n_tokens
        f_mean = jnp.sum(c_acc[...], axis=1, keepdims=True) / (n_tokens * _TOP_K)
        aux = _ALPHA * n_experts * jnp.sum(p_mean * f_mean)
        stats_ref[...] = jnp.broadcast_to(aux, (1, n_experts))


def kernel(hidden_states, gate_weight):
    b, s, h = hidden_states.shape
    e = gate_weight.shape[0]
    t = b * s
    x = hidden_states.reshape(t, h)

    body = functools.partial(_router_body, n_tokens=t, n_experts=e)
    wout, iout, stats = pl.pallas_call(
        body,
        grid=(t // _TB,),
        in_specs=[
            pl.BlockSpec((_TB, h), lambda i: (i, 0)),
            pl.BlockSpec((e, h), lambda i: (0, 0)),
        ],
        out_specs=[
            pl.BlockSpec((_TB, _TOP_K), lambda i: (i, 0)),
            pl.BlockSpec((_TB, _TOP_K), lambda i: (i, 0)),
            pl.BlockSpec((1, e), lambda i: (0, 0)),
        ],
        out_shape=[
            jax.ShapeDtypeStruct((t, _TOP_K), jnp.float32),
            jax.ShapeDtypeStruct((t, _TOP_K), jnp.int32),
            jax.ShapeDtypeStruct((1, e), jnp.float32),
        ],
        scratch_shapes=[
            pltpu.VMEM((e, _TB), jnp.float32),
            pltpu.VMEM((e, _TB), jnp.float32),
        ],
    )(x, gate_weight)

    return (
        wout.reshape(b, s, _TOP_K),
        iout.reshape(b, s, _TOP_K).astype(jnp.int64),
        stats[0, 0],
    )


# speedup vs baseline: 1.9106x; 1.0642x over previous
import functools

import jax
import jax.numpy as jnp
from jax import lax
from jax.experimental import pallas as pl
from jax.experimental.pallas import tpu as pltpu

_TOP_K = 2
_ALPHA = 0.01
_TB = 512


def _router_body(x_ref, w_ref, wout_ref, iout_ref, stats_ref,
                 logit_buf, p_acc, c_acc, *, n_tokens, n_experts, n_blocks):
    step = pl.program_id(0)

    prev = logit_buf[...]

    x = x_ref[...]
    w = w_ref[...]
    logit_buf[...] = lax.dot_general(
        w, x, (((1,), (1,)), ((), ())), preferred_element_type=jnp.float32
    )

    eidx = lax.broadcasted_iota(jnp.int32, prev.shape, 0)
    m1 = jnp.max(prev, axis=0, keepdims=True)
    i1 = jnp.min(jnp.where(prev == m1, eidx, n_experts), axis=0, keepdims=True)
    masked = jnp.where(eidx == i1, -jnp.inf, prev)
    m2 = jnp.max(masked, axis=0, keepdims=True)
    i2 = jnp.min(jnp.where(masked == m2, eidx, n_experts), axis=0, keepdims=True)

    ex = jnp.exp(prev - m1)
    z = jnp.sum(ex, axis=0, keepdims=True)
    p1 = 1.0 / z
    p2 = jnp.exp(m2 - m1) / z
    denom = p1 + p2 + 1e-9
    wout_ref[...] = jnp.concatenate([p1 / denom, p2 / denom], axis=0).T
    iout_ref[...] = jnp.concatenate([i1, i2], axis=0).T

    one_hot = (eidx == i1).astype(jnp.float32) + (eidx == i2).astype(jnp.float32)
    p_acc[...] += ex / z
    c_acc[...] += one_hot

    @pl.when(step == 0)
    def _init():
        p_acc[...] = jnp.zeros_like(p_acc)
        c_acc[...] = jnp.zeros_like(c_acc)

    @pl.when(step == n_blocks)
    def _finish():
        p_mean = jnp.sum(p_acc[...], axis=1, keepdims=True) / n_tokens
        f_mean = jnp.sum(c_acc[...], axis=1, keepdims=True) / (n_tokens * _TOP_K)
        aux = _ALPHA * n_experts * jnp.sum(p_mean * f_mean)
        stats_ref[...] = jnp.broadcast_to(aux, (1, n_experts))


def kernel(hidden_states, gate_weight):
    b, s, h = hidden_states.shape
    e = gate_weight.shape[0]
    t = b * s
    n_blocks = t // _TB

    x = hidden_states.reshape(t, h)
    body = functools.partial(
        _router_body, n_tokens=t, n_experts=e, n_blocks=n_blocks)
    wout, iout, stats = pl.pallas_call(
        body,
        grid=(n_blocks + 1,),
        in_specs=[
            pl.BlockSpec((_TB, h), lambda i: (jnp.minimum(i, n_blocks - 1), 0)),
            pl.BlockSpec((e, h), lambda i: (0, 0)),
        ],
        out_specs=[
            pl.BlockSpec((_TB, _TOP_K), lambda i: (jnp.maximum(i - 1, 0), 0)),
            pl.BlockSpec((_TB, _TOP_K), lambda i: (jnp.maximum(i - 1, 0), 0)),
            pl.BlockSpec((1, e), lambda i: (0, 0)),
        ],
        out_shape=[
            jax.ShapeDtypeStruct((t, _TOP_K), jnp.float32),
            jax.ShapeDtypeStruct((t, _TOP_K), jnp.int32),
            jax.ShapeDtypeStruct((1, e), jnp.float32),
        ],
        scratch_shapes=[
            pltpu.VMEM((e, _TB), jnp.float32),
            pltpu.VMEM((e, _TB), jnp.float32),
            pltpu.VMEM((e, _TB), jnp.float32),
        ],
    )(x, gate_weight)

    return (
        wout.reshape(b, s, _TOP_K),
        iout.reshape(b, s, _TOP_K).astype(jnp.int64),
        stats[0, 0],
    )


# speedup vs baseline: 2.1178x; 1.1085x over previous
import functools

import jax
import jax.numpy as jnp
from jax import lax
from jax.experimental import pallas as pl
from jax.experimental.pallas import tpu as pltpu

_TOP_K = 2
_ALPHA = 0.01
_TB = 1024


def _router_body(x_ref, w_ref, wout_ref, iout_ref, stats_ref,
                 logit_buf, p_acc, c_acc, *, n_tokens, n_experts, n_blocks):
    step = pl.program_id(0)

    prev = logit_buf[...]

    x = x_ref[...]
    w = w_ref[...]
    logit_buf[...] = lax.dot_general(
        w, x, (((1,), (1,)), ((), ())), preferred_element_type=jnp.float32
    )

    eidx = lax.broadcasted_iota(jnp.int32, prev.shape, 0)
    m1 = jnp.max(prev, axis=0, keepdims=True)
    i1 = jnp.min(jnp.where(prev == m1, eidx, n_experts), axis=0, keepdims=True)
    masked = jnp.where(eidx == i1, -jnp.inf, prev)
    m2 = jnp.max(masked, axis=0, keepdims=True)
    i2 = jnp.min(jnp.where(masked == m2, eidx, n_experts), axis=0, keepdims=True)

    ex = jnp.exp(prev - m1)
    z = jnp.sum(ex, axis=0, keepdims=True)
    p1 = 1.0 / z
    p2 = jnp.exp(m2 - m1) / z
    denom = p1 + p2 + 1e-9
    wout_ref[...] = jnp.concatenate([p1 / denom, p2 / denom], axis=0).T
    iout_ref[...] = jnp.concatenate([i1, i2], axis=0).T

    one_hot = (eidx == i1).astype(jnp.float32) + (eidx == i2).astype(jnp.float32)
    p_acc[...] += ex / z
    c_acc[...] += one_hot

    @pl.when(step == 0)
    def _init():
        p_acc[...] = jnp.zeros_like(p_acc)
        c_acc[...] = jnp.zeros_like(c_acc)

    @pl.when(step == n_blocks)
    def _finish():
        p_mean = jnp.sum(p_acc[...], axis=1, keepdims=True) / n_tokens
        f_mean = jnp.sum(c_acc[...], axis=1, keepdims=True) / (n_tokens * _TOP_K)
        aux = _ALPHA * n_experts * jnp.sum(p_mean * f_mean)
        stats_ref[...] = jnp.broadcast_to(aux, (1, n_experts))


def kernel(hidden_states, gate_weight):
    b, s, h = hidden_states.shape
    e = gate_weight.shape[0]
    t = b * s
    n_blocks = t // _TB

    x = hidden_states.reshape(t, h)
    body = functools.partial(
        _router_body, n_tokens=t, n_experts=e, n_blocks=n_blocks)
    wout, iout, stats = pl.pallas_call(
        body,
        grid=(n_blocks + 1,),
        in_specs=[
            pl.BlockSpec((_TB, h), lambda i: (jnp.minimum(i, n_blocks - 1), 0)),
            pl.BlockSpec((e, h), lambda i: (0, 0)),
        ],
        out_specs=[
            pl.BlockSpec((_TB, _TOP_K), lambda i: (jnp.maximum(i - 1, 0), 0)),
            pl.BlockSpec((_TB, _TOP_K), lambda i: (jnp.maximum(i - 1, 0), 0)),
            pl.BlockSpec((1, e), lambda i: (0, 0)),
        ],
        out_shape=[
            jax.ShapeDtypeStruct((t, _TOP_K), jnp.float32),
            jax.ShapeDtypeStruct((t, _TOP_K), jnp.int32),
            jax.ShapeDtypeStruct((1, e), jnp.float32),
        ],
        scratch_shapes=[
            pltpu.VMEM((e, _TB), jnp.float32),
            pltpu.VMEM((e, _TB), jnp.float32),
            pltpu.VMEM((e, _TB), jnp.float32),
        ],
    )(x, gate_weight)

    return (
        wout.reshape(b, s, _TOP_K),
        iout.reshape(b, s, _TOP_K).astype(jnp.int64),
        stats[0, 0],
    )
